# Initial kernel scaffold; baseline (speedup 1.0000x reference)
import jax
import jax.numpy as jnp
from jax.experimental import pallas as pl


def kernel(x, W):
    raise NotImplementedError("write your pallas kernel here")


# speedup vs baseline: 9.2869x; 9.2869x over previous
import functools

import jax
import jax.numpy as jnp
from jax import lax
from jax.experimental import pallas as pl
from jax.experimental.pallas import tpu as pltpu
from jax.experimental.pallas import tpu_sc as plsc

_COMMIT = 0.25


def _assign_body(xsq_ref, wsq_ref, x_ref, wt_ref, idx_ref, cnt_ref, acc_ref):
    i = pl.program_id(0)
    mb = x_ref.shape[0]
    ne = wt_ref.shape[1]
    mm = lax.dot_general(x_ref[...], wt_ref[...],
                         dimension_numbers=(((1,), (0,)), ((), ())),
                         preferred_element_type=jnp.float32)
    d = (xsq_ref[...] + wsq_ref[...]) - 2.0 * mm
    m = jnp.min(d, axis=1, keepdims=True)
    colf = lax.broadcasted_iota(jnp.int32, (mb, ne), 1).astype(jnp.float32)
    idxf = jnp.min(jnp.where(d == m, colf, jnp.float32(ne)), axis=1,
                   keepdims=True)
    idx_ref[0, 0, :] = idxf[:, 0].astype(jnp.int32)
    h = jnp.sum((colf == idxf).astype(jnp.float32), axis=0, keepdims=True)

    @pl.when(i == 0)
    def _():
        acc_ref[...] = jnp.zeros_like(acc_ref)

    acc_ref[...] += h

    @pl.when(i == pl.num_programs(0) - 1)
    def _():
        cnt_ref[...] = acc_ref[...]


def _assign(flat_x, xsq, wsq, wt, mb):
    n, ed = flat_x.shape
    ne = wt.shape[1]
    nb = n // mb
    idx3, cnt = pl.pallas_call(
        _assign_body,
        grid=(nb,),
        in_specs=[
            pl.BlockSpec((mb, 1), lambda i: (i, 0)),
            pl.BlockSpec((1, ne), lambda i: (0, 0)),
            pl.BlockSpec((mb, ed), lambda i: (i, 0)),
            pl.BlockSpec((ed, ne), lambda i: (0, 0)),
        ],
        out_specs=[
            pl.BlockSpec((1, 1, mb), lambda i: (i, 0, 0)),
            pl.BlockSpec((1, ne), lambda i: (0, 0)),
        ],
        out_shape=[
            jax.ShapeDtypeStruct((nb, 1, mb), jnp.int32),
            jax.ShapeDtypeStruct((1, ne), jnp.float32),
        ],
        scratch_shapes=[pltpu.VMEM((1, ne), jnp.float32)],
    )(xsq, wsq, flat_x, wt)
    return idx3.reshape(1, n), cnt


def _gather(w, idx2d, window=128):
    n = idx2d.shape[1]
    ed = w.shape[1]
    mesh = plsc.VectorSubcoreMesh(core_axis_name="core",
                                  subcore_axis_name="subcore")

    @functools.partial(
        pl.kernel,
        out_type=jax.ShapeDtypeStruct((n, ed), jnp.float32),
        mesh=mesh)
    def k(w_hbm, i_hbm, o_hbm):
        def body(i_vmem, o_vmem):
            pltpu.sync_copy(w_hbm.at[i_vmem.at[0]], o_vmem)

        pltpu.emit_pipeline(
            body,
            grid=(n // window,),
            in_specs=[pl.BlockSpec((1, window), index_map=lambda i: (0, i))],
            out_specs=[pl.BlockSpec((window, ed), index_map=lambda i: (i, 0))],
            core_axis_name=("core", "subcore"),
            dimension_semantics=(pltpu.PARALLEL,),
        )(i_hbm, o_hbm)

    return k(w, idx2d)


def _finalize_body(q_ref, x_ref, cnt_ref, st_ref, loss_ref, perp_ref, acc_ref):
    i = pl.program_id(0)
    diff = q_ref[...] - x_ref[...]
    st_ref[...] = x_ref[...] + diff
    s = jnp.sum(diff * diff, keepdims=True).reshape(1, 1)

    @pl.when(i == 0)
    def _():
        acc_ref[...] = jnp.zeros_like(acc_ref)

    acc_ref[...] += s

    @pl.when(i == pl.num_programs(0) - 1)
    def _():
        nrows = pl.num_programs(0) * q_ref.shape[0]
        v = acc_ref[...] / jnp.float32(nrows * q_ref.shape[1])
        loss_ref[...] = v + jnp.float32(_COMMIT) * v
        p = cnt_ref[...] / jnp.float32(nrows)
        ent = p * jnp.log(p + jnp.float32(1e-10))
        perp_ref[...] = jnp.exp(-jnp.sum(ent, keepdims=True).reshape(1, 1))


def _finalize(q, flat_x, cnt, mb):
    n, ed = flat_x.shape
    ne = cnt.shape[1]
    nb = n // mb
    st, loss, perp = pl.pallas_call(
        _finalize_body,
        grid=(nb,),
        in_specs=[
            pl.BlockSpec((mb, ed), lambda i: (i, 0)),
            pl.BlockSpec((mb, ed), lambda i: (i, 0)),
            pl.BlockSpec((1, ne), lambda i: (0, 0)),
        ],
        out_specs=[
            pl.BlockSpec((mb, ed), lambda i: (i, 0)),
            pl.BlockSpec((1, 1), lambda i: (0, 0)),
            pl.BlockSpec((1, 1), lambda i: (0, 0)),
        ],
        out_shape=[
            jax.ShapeDtypeStruct((n, ed), jnp.float32),
            jax.ShapeDtypeStruct((1, 1), jnp.float32),
            jax.ShapeDtypeStruct((1, 1), jnp.float32),
        ],
        scratch_shapes=[pltpu.VMEM((1, 1), jnp.float32)],
    )(q, flat_x, cnt)
    return st, loss, perp


def kernel(x, W):
    ed = W.shape[1]
    flat_x = x.reshape(-1, ed)
    xsq = jnp.sum(flat_x ** 2, axis=1, keepdims=True)
    wsq = jnp.sum(W ** 2, axis=1).reshape(1, -1)
    wt = W.T
    mb = 256
    idx2d, cnt = _assign(flat_x, xsq, wsq, wt, mb)
    q = _gather(W, idx2d)
    st, loss, perp = _finalize(q, flat_x, cnt, mb)
    return (st.reshape(x.shape), loss.reshape(()), perp.reshape(()))
